# baseline (speedup 1.0000x reference)
import functools

import jax
import jax.numpy as jnp
from jax import lax
from jax.experimental import pallas as pl
from jax.experimental.pallas import tpu as pltpu
from jax.experimental.pallas import tpu_sc as plsc

N = 10000
NPAD = 10240
E = 320000
DIN = 128
DOUT = 64
K_HOPS = 3

NC = 2
NS = 16
NW = NC * NS
CH = 128
CPW0 = 80
CPW1 = 80
EPAD = NS * CH * (CPW0 + CPW1)
ROWS_PER_TILE = NPAD // NS

_mesh = plsc.VectorSubcoreMesh(core_axis_name="c", subcore_axis_name="s")
_sc_params = pltpu.CompilerParams(use_tc_tiling_on_sc=False)


@functools.partial(
    pl.kernel,
    out_type=jax.ShapeDtypeStruct((NC * NPAD, 16), jnp.float32),
    mesh=_mesh,
    scratch_types=[
        pltpu.VMEM((CPW0, CH), jnp.int32),
        pltpu.VMEM((CH, 16), jnp.float32),
        pltpu.VMEM((ROWS_PER_TILE, 16), jnp.float32),
        pltpu.VMEM_SHARED((NPAD, 16), jnp.float32),
    ],
    compiler_params=_sc_params,
)
def _sc_degree(dst0_hbm, dst1_hbm, out_hbm, dst_v, obuf, zbuf, acc_sp):
    cid = lax.axis_index("c")
    sid = lax.axis_index("s")

    @pl.when(cid == 0)
    def _():
        pltpu.sync_copy(dst0_hbm.at[sid], dst_v.at[pl.ds(0, CPW0)])

    @pl.when(cid == 1)
    def _():
        pltpu.sync_copy(dst1_hbm.at[sid], dst_v.at[pl.ds(0, CPW1)])

    nch = jnp.where(cid == 0, CPW0, CPW1)
    one_hot = jnp.where(lax.iota(jnp.int32, 16) == 0,
                        jnp.float32(1.0), jnp.float32(0.0))
    zeros16 = jnp.zeros((16,), jnp.float32)

    def fill_obuf(r, _):
        obuf[r, :] = one_hot
        return 0

    lax.fori_loop(0, CH, fill_obuf, 0)

    def fill_z(r, _):
        zbuf[r, :] = zeros16
        return 0

    lax.fori_loop(0, ROWS_PER_TILE, fill_z, 0)
    pltpu.sync_copy(zbuf, acc_sp.at[pl.ds(sid * ROWS_PER_TILE, ROWS_PER_TILE)])
    plsc.subcore_barrier()

    def chunk(ci, _):
        pltpu.sync_copy(obuf, acc_sp.at[dst_v.at[ci]], add=True)
        return 0

    lax.fori_loop(0, nch, chunk, 0)
    plsc.subcore_barrier()
    base = cid * NPAD + sid * ROWS_PER_TILE
    pltpu.sync_copy(acc_sp.at[pl.ds(sid * ROWS_PER_TILE, ROWS_PER_TILE)],
                    out_hbm.at[pl.ds(base, ROWS_PER_TILE)])


@functools.partial(
    pl.kernel,
    out_type=jax.ShapeDtypeStruct((NC * NPAD, DOUT), jnp.float32),
    mesh=_mesh,
    scratch_types=[
        pltpu.VMEM((CPW0, CH), jnp.int32),
        pltpu.VMEM((CPW0, CH), jnp.int32),
        pltpu.VMEM((2, CH, DOUT), jnp.float32),
        pltpu.VMEM((ROWS_PER_TILE, DOUT), jnp.float32),
        pltpu.VMEM_SHARED((NPAD, DOUT), jnp.float32),
        pltpu.SemaphoreType.DMA,
        pltpu.SemaphoreType.DMA,
    ],
    compiler_params=_sc_params,
)
def _sc_hop(h_hbm, src0_hbm, dst0_hbm, src1_hbm, dst1_hbm, out_hbm,
            src_v, dst_v, gbuf, zbuf, acc_sp, sem0, sem1):
    cid = lax.axis_index("c")
    sid = lax.axis_index("s")

    @pl.when(cid == 0)
    def _():
        pltpu.sync_copy(src0_hbm.at[sid], src_v.at[pl.ds(0, CPW0)])
        pltpu.sync_copy(dst0_hbm.at[sid], dst_v.at[pl.ds(0, CPW0)])

    @pl.when(cid == 1)
    def _():
        pltpu.sync_copy(src1_hbm.at[sid], src_v.at[pl.ds(0, CPW1)])
        pltpu.sync_copy(dst1_hbm.at[sid], dst_v.at[pl.ds(0, CPW1)])

    nch = jnp.where(cid == 0, CPW0, CPW1)
    zeros16 = jnp.zeros((16,), jnp.float32)

    def fill_z(r, _):
        for j in range(DOUT // 16):
            zbuf[r, pl.ds(j * 16, 16)] = zeros16
        return 0

    lax.fori_loop(0, ROWS_PER_TILE, fill_z, 0)
    pltpu.sync_copy(zbuf, acc_sp.at[pl.ds(sid * ROWS_PER_TILE, ROWS_PER_TILE)])
    plsc.subcore_barrier()

    def chunk(ci, _):
        pltpu.sync_copy(h_hbm.at[src_v.at[ci]], gbuf.at[0])
        pltpu.sync_copy(gbuf.at[0], acc_sp.at[dst_v.at[ci]], add=True)
        return 0

    lax.fori_loop(0, nch, chunk, 0)
    plsc.subcore_barrier()
    base = cid * NPAD + sid * ROWS_PER_TILE
    pltpu.sync_copy(acc_sp.at[pl.ds(sid * ROWS_PER_TILE, ROWS_PER_TILE)],
                    out_hbm.at[pl.ds(base, ROWS_PER_TILE)])


def _tc_w_body(w1_ref, w2_ref, b1_ref, b2_ref, w_ref, bv_ref):
    w_ref[...] = jnp.dot(w1_ref[...], w2_ref[...],
                         preferred_element_type=jnp.float32)
    bv_ref[...] = jnp.dot(b1_ref[...], w2_ref[...],
                          preferred_element_type=jnp.float32) + b2_ref[...]


_tc_w = pl.pallas_call(
    _tc_w_body,
    out_shape=(
        jax.ShapeDtypeStruct((DIN, DOUT), jnp.float32),
        jax.ShapeDtypeStruct((8, DOUT), jnp.float32),
    ),
)


def _tc_h0_body(x_ref, w_ref, dp_ref, o_ref):
    deg = dp_ref[0, :, 0] + dp_ref[1, :, 0] + 1.0
    s = lax.rsqrt(deg)
    o_ref[...] = jnp.dot(x_ref[...], w_ref[...],
                         preferred_element_type=jnp.float32) * s[:, None]


_tc_h0 = pl.pallas_call(
    _tc_h0_body,
    grid=(NPAD // 128,),
    in_specs=[
        pl.BlockSpec((128, DIN), lambda i: (i, 0)),
        pl.BlockSpec((DIN, DOUT), lambda i: (0, 0)),
        pl.BlockSpec((2, 128, 16), lambda i: (0, i, 0)),
    ],
    out_specs=pl.BlockSpec((128, DOUT), lambda i: (i, 0)),
    out_shape=jax.ShapeDtypeStruct((NPAD, DOUT), jnp.float32),
)


def _tc_comb_body(pp_ref, h_ref, dp_ref, o_ref):
    deg = dp_ref[0, :, 0] + dp_ref[1, :, 0] + 1.0
    d = 1.0 / deg
    o_ref[...] = (pp_ref[0] + pp_ref[1] + h_ref[...]) * d[:, None]


_tc_comb = pl.pallas_call(
    _tc_comb_body,
    grid=(NPAD // 128,),
    in_specs=[
        pl.BlockSpec((2, 128, DOUT), lambda i: (0, i, 0)),
        pl.BlockSpec((128, DOUT), lambda i: (i, 0)),
        pl.BlockSpec((2, 128, 16), lambda i: (0, i, 0)),
    ],
    out_specs=pl.BlockSpec((128, DOUT), lambda i: (i, 0)),
    out_shape=jax.ShapeDtypeStruct((NPAD, DOUT), jnp.float32),
)


def _tc_final_body(pp_ref, h_ref, dp_ref, bv_ref, o_ref):
    deg = dp_ref[0, :, 0] + dp_ref[1, :, 0] + 1.0
    s = lax.rsqrt(deg)
    o_ref[...] = ((pp_ref[0] + pp_ref[1] + h_ref[...]) * s[:, None]
                  + bv_ref[0:1, :])


_tc_final = pl.pallas_call(
    _tc_final_body,
    grid=(NPAD // 128,),
    in_specs=[
        pl.BlockSpec((2, 128, DOUT), lambda i: (0, i, 0)),
        pl.BlockSpec((128, DOUT), lambda i: (i, 0)),
        pl.BlockSpec((2, 128, 16), lambda i: (0, i, 0)),
        pl.BlockSpec((8, DOUT), lambda i: (0, 0)),
    ],
    out_specs=pl.BlockSpec((128, DOUT), lambda i: (i, 0)),
    out_shape=jax.ShapeDtypeStruct((NPAD, DOUT), jnp.float32),
)


@jax.jit
def kernel(x, edge_index, W1, b1, W2, b2):
    src = edge_index[0].astype(jnp.int32)
    dst = edge_index[1].astype(jnp.int32)
    pad = jnp.full((EPAD - E,), N, dtype=jnp.int32)
    srcf = jnp.concatenate([src, pad])
    dstf = jnp.concatenate([dst, pad])
    n0 = NS * CPW0 * CH
    src0 = srcf[:n0].reshape(NS, CPW0, CH)
    dst0 = dstf[:n0].reshape(NS, CPW0, CH)
    src1 = srcf[n0:].reshape(NS, CPW1, CH)
    dst1 = dstf[n0:].reshape(NS, CPW1, CH)
    xp = jnp.pad(x, ((0, NPAD - N), (0, 0)))
    b1r = jnp.broadcast_to(b1[None, :], (8, DIN))
    b2r = jnp.broadcast_to(b2[None, :], (8, DOUT))

    degp = _sc_degree(dst0, dst1).reshape(NC, NPAD, 16)
    w_f, bv = _tc_w(W1, W2, b1r, b2r)
    h = _tc_h0(xp, w_f, degp)
    for hop in range(K_HOPS):
        pp = _sc_hop(h, src0, dst0, src1, dst1).reshape(NC, NPAD, DOUT)
        if hop < K_HOPS - 1:
            h = _tc_comb(pp, h, degp)
        else:
            h = _tc_final(pp, h, degp, bv)
    return h[:N]


# speedup vs baseline: 2.4980x; 2.4980x over previous
import functools

import jax
import jax.numpy as jnp
from jax import lax
from jax.experimental import pallas as pl
from jax.experimental.pallas import tpu as pltpu
from jax.experimental.pallas import tpu_sc as plsc

N = 10000
NPAD = 10240
E = 320000
DIN = 128
DOUT = 64
K_HOPS = 3

NC = 2
NS = 16
NW = NC * NS
CH = 128
CPW0 = 80
CPW1 = 80
EPAD = NS * CH * (CPW0 + CPW1)
ROWS_PER_TILE = NPAD // NS

_mesh = plsc.VectorSubcoreMesh(core_axis_name="c", subcore_axis_name="s")
_sc_params = pltpu.CompilerParams(use_tc_tiling_on_sc=False)


@functools.partial(
    pl.kernel,
    out_type=jax.ShapeDtypeStruct((NC * NPAD, 16), jnp.float32),
    mesh=_mesh,
    scratch_types=[
        pltpu.VMEM((CPW0, CH), jnp.int32),
        pltpu.VMEM((CH, 16), jnp.float32),
        pltpu.VMEM((ROWS_PER_TILE, 16), jnp.float32),
        pltpu.VMEM_SHARED((NPAD, 16), jnp.float32),
    ],
    compiler_params=_sc_params,
)
def _sc_degree(dst0_hbm, dst1_hbm, out_hbm, dst_v, obuf, zbuf, acc_sp):
    cid = lax.axis_index("c")
    sid = lax.axis_index("s")

    @pl.when(cid == 0)
    def _():
        pltpu.sync_copy(dst0_hbm.at[sid], dst_v.at[pl.ds(0, CPW0)])

    @pl.when(cid == 1)
    def _():
        pltpu.sync_copy(dst1_hbm.at[sid], dst_v.at[pl.ds(0, CPW1)])

    nch = jnp.where(cid == 0, CPW0, CPW1)
    one_hot = jnp.where(lax.iota(jnp.int32, 16) == 0,
                        jnp.float32(1.0), jnp.float32(0.0))
    zeros16 = jnp.zeros((16,), jnp.float32)

    def fill_obuf(r, _):
        obuf[r, :] = one_hot
        return 0

    lax.fori_loop(0, CH, fill_obuf, 0)

    def fill_z(r, _):
        zbuf[r, :] = zeros16
        return 0

    lax.fori_loop(0, ROWS_PER_TILE, fill_z, 0)
    pltpu.sync_copy(zbuf, acc_sp.at[pl.ds(sid * ROWS_PER_TILE, ROWS_PER_TILE)])
    plsc.subcore_barrier()

    def chunk(ci, _):
        pltpu.sync_copy(obuf, acc_sp.at[dst_v.at[ci]], add=True)
        return 0

    lax.fori_loop(0, nch, chunk, 0)
    plsc.subcore_barrier()
    base = cid * NPAD + sid * ROWS_PER_TILE
    pltpu.sync_copy(acc_sp.at[pl.ds(sid * ROWS_PER_TILE, ROWS_PER_TILE)],
                    out_hbm.at[pl.ds(base, ROWS_PER_TILE)])


@functools.partial(
    pl.kernel,
    out_type=jax.ShapeDtypeStruct((NC * NPAD, DOUT), jnp.float32),
    mesh=_mesh,
    scratch_types=[
        pltpu.VMEM((CPW0, CH), jnp.int32),
        pltpu.VMEM((CPW0, CH), jnp.int32),
        pltpu.VMEM((2, CH, DOUT), jnp.float32),
        pltpu.VMEM((ROWS_PER_TILE, DOUT), jnp.float32),
        pltpu.VMEM_SHARED((NPAD, DOUT), jnp.float32),
        pltpu.SemaphoreType.DMA,
        pltpu.SemaphoreType.DMA,
    ],
    compiler_params=_sc_params,
)
def _sc_hop(h_hbm, src0_hbm, dst0_hbm, src1_hbm, dst1_hbm, out_hbm,
            src_v, dst_v, gbuf, zbuf, acc_sp, sem0, sem1):
    cid = lax.axis_index("c")
    sid = lax.axis_index("s")

    @pl.when(cid == 0)
    def _():
        pltpu.sync_copy(src0_hbm.at[sid], src_v.at[pl.ds(0, CPW0)])
        pltpu.sync_copy(dst0_hbm.at[sid], dst_v.at[pl.ds(0, CPW0)])

    @pl.when(cid == 1)
    def _():
        pltpu.sync_copy(src1_hbm.at[sid], src_v.at[pl.ds(0, CPW1)])
        pltpu.sync_copy(dst1_hbm.at[sid], dst_v.at[pl.ds(0, CPW1)])

    nch = jnp.where(cid == 0, CPW0, CPW1)
    zeros16 = jnp.zeros((16,), jnp.float32)

    def fill_z(r, _):
        for j in range(DOUT // 16):
            zbuf[r, pl.ds(j * 16, 16)] = zeros16
        return 0

    lax.fori_loop(0, ROWS_PER_TILE, fill_z, 0)
    pltpu.sync_copy(zbuf, acc_sp.at[pl.ds(sid * ROWS_PER_TILE, ROWS_PER_TILE)])
    plsc.subcore_barrier()

    sems = (sem0, sem1)
    pltpu.async_copy(h_hbm.at[src_v.at[0]], gbuf.at[0], sem0)
    pltpu.async_copy(h_hbm.at[src_v.at[1]], gbuf.at[1], sem1)

    def outer(j, _):
        for b in range(2):
            ci = j * 2 + b
            pltpu.make_async_copy(h_hbm.at[src_v.at[ci]], gbuf.at[b],
                                  sems[b]).wait()
            pltpu.sync_copy(gbuf.at[b], acc_sp.at[dst_v.at[ci]], add=True)

            @pl.when(ci + 2 < nch)
            def _():
                pltpu.async_copy(h_hbm.at[src_v.at[ci + 2]], gbuf.at[b],
                                 sems[b])
        return 0

    lax.fori_loop(0, nch // 2, outer, 0)
    plsc.subcore_barrier()
    base = cid * NPAD + sid * ROWS_PER_TILE
    pltpu.sync_copy(acc_sp.at[pl.ds(sid * ROWS_PER_TILE, ROWS_PER_TILE)],
                    out_hbm.at[pl.ds(base, ROWS_PER_TILE)])


def _tc_w_body(w1_ref, w2_ref, b1_ref, b2_ref, w_ref, bv_ref):
    w_ref[...] = jnp.dot(w1_ref[...], w2_ref[...],
                         preferred_element_type=jnp.float32)
    bv_ref[...] = jnp.dot(b1_ref[...], w2_ref[...],
                          preferred_element_type=jnp.float32) + b2_ref[...]


_tc_w = pl.pallas_call(
    _tc_w_body,
    out_shape=(
        jax.ShapeDtypeStruct((DIN, DOUT), jnp.float32),
        jax.ShapeDtypeStruct((8, DOUT), jnp.float32),
    ),
)


def _tc_h0_body(x_ref, w_ref, dp_ref, o_ref):
    deg = dp_ref[0, :, 0] + dp_ref[1, :, 0] + 1.0
    s = lax.rsqrt(deg)
    o_ref[...] = jnp.dot(x_ref[...], w_ref[...],
                         preferred_element_type=jnp.float32) * s[:, None]


_tc_h0 = pl.pallas_call(
    _tc_h0_body,
    grid=(NPAD // 128,),
    in_specs=[
        pl.BlockSpec((128, DIN), lambda i: (i, 0)),
        pl.BlockSpec((DIN, DOUT), lambda i: (0, 0)),
        pl.BlockSpec((2, 128, 16), lambda i: (0, i, 0)),
    ],
    out_specs=pl.BlockSpec((128, DOUT), lambda i: (i, 0)),
    out_shape=jax.ShapeDtypeStruct((NPAD, DOUT), jnp.float32),
)


def _tc_comb_body(pp_ref, h_ref, dp_ref, o_ref):
    deg = dp_ref[0, :, 0] + dp_ref[1, :, 0] + 1.0
    d = 1.0 / deg
    o_ref[...] = (pp_ref[0] + pp_ref[1] + h_ref[...]) * d[:, None]


_tc_comb = pl.pallas_call(
    _tc_comb_body,
    grid=(NPAD // 128,),
    in_specs=[
        pl.BlockSpec((2, 128, DOUT), lambda i: (0, i, 0)),
        pl.BlockSpec((128, DOUT), lambda i: (i, 0)),
        pl.BlockSpec((2, 128, 16), lambda i: (0, i, 0)),
    ],
    out_specs=pl.BlockSpec((128, DOUT), lambda i: (i, 0)),
    out_shape=jax.ShapeDtypeStruct((NPAD, DOUT), jnp.float32),
)


def _tc_final_body(pp_ref, h_ref, dp_ref, bv_ref, o_ref):
    deg = dp_ref[0, :, 0] + dp_ref[1, :, 0] + 1.0
    s = lax.rsqrt(deg)
    o_ref[...] = ((pp_ref[0] + pp_ref[1] + h_ref[...]) * s[:, None]
                  + bv_ref[0:1, :])


_tc_final = pl.pallas_call(
    _tc_final_body,
    grid=(NPAD // 128,),
    in_specs=[
        pl.BlockSpec((2, 128, DOUT), lambda i: (0, i, 0)),
        pl.BlockSpec((128, DOUT), lambda i: (i, 0)),
        pl.BlockSpec((2, 128, 16), lambda i: (0, i, 0)),
        pl.BlockSpec((8, DOUT), lambda i: (0, 0)),
    ],
    out_specs=pl.BlockSpec((128, DOUT), lambda i: (i, 0)),
    out_shape=jax.ShapeDtypeStruct((NPAD, DOUT), jnp.float32),
)


@jax.jit
def kernel(x, edge_index, W1, b1, W2, b2):
    src = edge_index[0].astype(jnp.int32)
    dst = edge_index[1].astype(jnp.int32)
    ar = jnp.arange(EPAD - E, dtype=jnp.int32)
    pad_src = ar % NPAD
    pad_dst = N + (ar % (NPAD - N))
    srcf = jnp.concatenate([src, pad_src])
    dstf = jnp.concatenate([dst, pad_dst])
    n0 = NS * CPW0 * CH
    src0 = srcf[:n0].reshape(NS, CPW0, CH)
    dst0 = dstf[:n0].reshape(NS, CPW0, CH)
    src1 = srcf[n0:].reshape(NS, CPW1, CH)
    dst1 = dstf[n0:].reshape(NS, CPW1, CH)
    xp = jnp.pad(x, ((0, NPAD - N), (0, 0)))
    b1r = jnp.broadcast_to(b1[None, :], (8, DIN))
    b2r = jnp.broadcast_to(b2[None, :], (8, DOUT))

    degp = _sc_degree(dst0, dst1).reshape(NC, NPAD, 16)
    w_f, bv = _tc_w(W1, W2, b1r, b2r)
    h = _tc_h0(xp, w_f, degp)
    for hop in range(K_HOPS):
        pp = _sc_hop(h, src0, dst0, src1, dst1).reshape(NC, NPAD, DOUT)
        if hop < K_HOPS - 1:
            h = _tc_comb(pp, h, degp)
        else:
            h = _tc_final(pp, h, degp, bv)
    return h[:N]


# speedup vs baseline: 2.5903x; 1.0370x over previous
import functools

import jax
import jax.numpy as jnp
from jax import lax
from jax.experimental import pallas as pl
from jax.experimental.pallas import tpu as pltpu
from jax.experimental.pallas import tpu_sc as plsc

N = 10000
NPAD = 10240
E = 320000
DIN = 128
DOUT = 64
K_HOPS = 3

NC = 2
NS = 16
NW = NC * NS
CH = 128
CPW0 = 80
CPW1 = 80
EPAD = NS * CH * (CPW0 + CPW1)
ROWS_PER_TILE = NPAD // NS

_mesh = plsc.VectorSubcoreMesh(core_axis_name="c", subcore_axis_name="s")
_sc_params = pltpu.CompilerParams(use_tc_tiling_on_sc=False)


@functools.partial(
    pl.kernel,
    out_type=jax.ShapeDtypeStruct((NC * NPAD, 16), jnp.float32),
    mesh=_mesh,
    scratch_types=[
        pltpu.VMEM((CPW0, CH), jnp.int32),
        pltpu.VMEM((CH, 16), jnp.float32),
        pltpu.VMEM((ROWS_PER_TILE, 16), jnp.float32),
        pltpu.VMEM_SHARED((NPAD, 16), jnp.float32),
    ],
    compiler_params=_sc_params,
)
def _sc_degree(dst0_hbm, dst1_hbm, out_hbm, dst_v, obuf, zbuf, acc_sp):
    cid = lax.axis_index("c")
    sid = lax.axis_index("s")

    @pl.when(cid == 0)
    def _():
        pltpu.sync_copy(dst0_hbm.at[sid], dst_v.at[pl.ds(0, CPW0)])

    @pl.when(cid == 1)
    def _():
        pltpu.sync_copy(dst1_hbm.at[sid], dst_v.at[pl.ds(0, CPW1)])

    nch = jnp.where(cid == 0, CPW0, CPW1)
    one_hot = jnp.where(lax.iota(jnp.int32, 16) == 0,
                        jnp.float32(1.0), jnp.float32(0.0))
    zeros16 = jnp.zeros((16,), jnp.float32)

    def fill_obuf(r, _):
        obuf[r, :] = one_hot
        return 0

    lax.fori_loop(0, CH, fill_obuf, 0)

    def fill_z(r, _):
        zbuf[r, :] = zeros16
        return 0

    lax.fori_loop(0, ROWS_PER_TILE, fill_z, 0)
    pltpu.sync_copy(zbuf, acc_sp.at[pl.ds(sid * ROWS_PER_TILE, ROWS_PER_TILE)])
    plsc.subcore_barrier()

    def chunk(ci, _):
        pltpu.sync_copy(obuf, acc_sp.at[dst_v.at[ci]], add=True)
        return 0

    lax.fori_loop(0, nch, chunk, 0)
    plsc.subcore_barrier()
    base = cid * NPAD + sid * ROWS_PER_TILE
    pltpu.sync_copy(acc_sp.at[pl.ds(sid * ROWS_PER_TILE, ROWS_PER_TILE)],
                    out_hbm.at[pl.ds(base, ROWS_PER_TILE)])


@functools.partial(
    pl.kernel,
    out_type=jax.ShapeDtypeStruct((NC * NPAD, DOUT), jnp.float32),
    mesh=_mesh,
    scratch_types=[
        pltpu.VMEM((CPW0, CH), jnp.int32),
        pltpu.VMEM((CPW0, CH), jnp.int32),
        pltpu.VMEM((4, CH, DOUT), jnp.float32),
        pltpu.VMEM((ROWS_PER_TILE // 4, DOUT), jnp.float32),
        pltpu.VMEM_SHARED((NPAD, DOUT), jnp.float32),
        pltpu.SemaphoreType.DMA,
        pltpu.SemaphoreType.DMA,
        pltpu.SemaphoreType.DMA,
        pltpu.SemaphoreType.DMA,
    ],
    compiler_params=_sc_params,
)
def _sc_hop(h_hbm, src0_hbm, dst0_hbm, src1_hbm, dst1_hbm, out_hbm,
            src_v, dst_v, gbuf, zbuf, acc_sp, sem0, sem1, sem2, sem3):
    cid = lax.axis_index("c")
    sid = lax.axis_index("s")

    @pl.when(cid == 0)
    def _():
        pltpu.sync_copy(src0_hbm.at[sid], src_v.at[pl.ds(0, CPW0)])
        pltpu.sync_copy(dst0_hbm.at[sid], dst_v.at[pl.ds(0, CPW0)])

    @pl.when(cid == 1)
    def _():
        pltpu.sync_copy(src1_hbm.at[sid], src_v.at[pl.ds(0, CPW1)])
        pltpu.sync_copy(dst1_hbm.at[sid], dst_v.at[pl.ds(0, CPW1)])

    nch = jnp.where(cid == 0, CPW0, CPW1)
    zeros16 = jnp.zeros((16,), jnp.float32)
    qrows = ROWS_PER_TILE // 4

    def fill_z(r, _):
        for j in range(DOUT // 16):
            zbuf[r, pl.ds(j * 16, 16)] = zeros16
        return 0

    lax.fori_loop(0, qrows, fill_z, 0)
    for q in range(4):
        pltpu.sync_copy(
            zbuf, acc_sp.at[pl.ds(sid * ROWS_PER_TILE + q * qrows, qrows)])
    plsc.subcore_barrier()

    sems = (sem0, sem1, sem2, sem3)
    for b in range(4):
        pltpu.async_copy(h_hbm.at[src_v.at[b]], gbuf.at[b], sems[b])

    def outer(j, _):
        for b in range(4):
            ci = j * 4 + b
            pltpu.make_async_copy(h_hbm.at[src_v.at[ci]], gbuf.at[b],
                                  sems[b]).wait()
            pltpu.sync_copy(gbuf.at[b], acc_sp.at[dst_v.at[ci]], add=True)

            @pl.when(ci + 4 < nch)
            def _():
                pltpu.async_copy(h_hbm.at[src_v.at[ci + 4]], gbuf.at[b],
                                 sems[b])
        return 0

    lax.fori_loop(0, nch // 4, outer, 0)
    plsc.subcore_barrier()
    base = cid * NPAD + sid * ROWS_PER_TILE
    pltpu.sync_copy(acc_sp.at[pl.ds(sid * ROWS_PER_TILE, ROWS_PER_TILE)],
                    out_hbm.at[pl.ds(base, ROWS_PER_TILE)])


def _tc_w_body(w1_ref, w2_ref, b1_ref, b2_ref, w_ref, bv_ref):
    w_ref[...] = jnp.dot(w1_ref[...], w2_ref[...],
                         preferred_element_type=jnp.float32)
    bv_ref[...] = jnp.dot(b1_ref[...], w2_ref[...],
                          preferred_element_type=jnp.float32) + b2_ref[...]


_tc_w = pl.pallas_call(
    _tc_w_body,
    out_shape=(
        jax.ShapeDtypeStruct((DIN, DOUT), jnp.float32),
        jax.ShapeDtypeStruct((8, DOUT), jnp.float32),
    ),
)


def _tc_xw_body(x_ref, w_ref, o_ref):
    o_ref[...] = jnp.dot(x_ref[...], w_ref[...],
                         preferred_element_type=jnp.float32)


_tc_xw = pl.pallas_call(
    _tc_xw_body,
    grid=(NPAD // 128,),
    in_specs=[
        pl.BlockSpec((128, DIN), lambda i: (i, 0)),
        pl.BlockSpec((DIN, DOUT), lambda i: (0, 0)),
    ],
    out_specs=pl.BlockSpec((128, DOUT), lambda i: (i, 0)),
    out_shape=jax.ShapeDtypeStruct((NPAD, DOUT), jnp.float32),
)


def _tc_scale_body(xw_ref, dp_ref, o_ref):
    deg = dp_ref[0, :, 0] + dp_ref[1, :, 0] + 1.0
    s = lax.rsqrt(deg)
    o_ref[...] = xw_ref[...] * s[:, None]


_tc_scale = pl.pallas_call(
    _tc_scale_body,
    grid=(NPAD // 128,),
    in_specs=[
        pl.BlockSpec((128, DOUT), lambda i: (i, 0)),
        pl.BlockSpec((2, 128, 16), lambda i: (0, i, 0)),
    ],
    out_specs=pl.BlockSpec((128, DOUT), lambda i: (i, 0)),
    out_shape=jax.ShapeDtypeStruct((NPAD, DOUT), jnp.float32),
)


def _tc_comb_body(pp_ref, h_ref, dp_ref, o_ref):
    deg = dp_ref[0, :, 0] + dp_ref[1, :, 0] + 1.0
    d = 1.0 / deg
    o_ref[...] = (pp_ref[0] + pp_ref[1] + h_ref[...]) * d[:, None]


_tc_comb = pl.pallas_call(
    _tc_comb_body,
    grid=(NPAD // 128,),
    in_specs=[
        pl.BlockSpec((2, 128, DOUT), lambda i: (0, i, 0)),
        pl.BlockSpec((128, DOUT), lambda i: (i, 0)),
        pl.BlockSpec((2, 128, 16), lambda i: (0, i, 0)),
    ],
    out_specs=pl.BlockSpec((128, DOUT), lambda i: (i, 0)),
    out_shape=jax.ShapeDtypeStruct((NPAD, DOUT), jnp.float32),
)


def _tc_final_body(pp_ref, h_ref, dp_ref, bv_ref, o_ref):
    deg = dp_ref[0, :, 0] + dp_ref[1, :, 0] + 1.0
    s = lax.rsqrt(deg)
    o_ref[...] = ((pp_ref[0] + pp_ref[1] + h_ref[...]) * s[:, None]
                  + bv_ref[0:1, :])


_tc_final = pl.pallas_call(
    _tc_final_body,
    grid=(NPAD // 128,),
    in_specs=[
        pl.BlockSpec((2, 128, DOUT), lambda i: (0, i, 0)),
        pl.BlockSpec((128, DOUT), lambda i: (i, 0)),
        pl.BlockSpec((2, 128, 16), lambda i: (0, i, 0)),
        pl.BlockSpec((8, DOUT), lambda i: (0, 0)),
    ],
    out_specs=pl.BlockSpec((128, DOUT), lambda i: (i, 0)),
    out_shape=jax.ShapeDtypeStruct((NPAD, DOUT), jnp.float32),
)


@jax.jit
def kernel(x, edge_index, W1, b1, W2, b2):
    src = edge_index[0].astype(jnp.int32)
    dst = edge_index[1].astype(jnp.int32)
    ar = jnp.arange(EPAD - E, dtype=jnp.int32)
    pad_src = ar % NPAD
    pad_dst = N + (ar % (NPAD - N))
    srcf = jnp.concatenate([src, pad_src])
    dstf = jnp.concatenate([dst, pad_dst])
    n0 = NS * CPW0 * CH
    src0 = srcf[:n0].reshape(NS, CPW0, CH)
    dst0 = dstf[:n0].reshape(NS, CPW0, CH)
    src1 = srcf[n0:].reshape(NS, CPW1, CH)
    dst1 = dstf[n0:].reshape(NS, CPW1, CH)
    xp = jnp.pad(x, ((0, NPAD - N), (0, 0)))
    b1r = jnp.broadcast_to(b1[None, :], (8, DIN))
    b2r = jnp.broadcast_to(b2[None, :], (8, DOUT))

    w_f, bv = _tc_w(W1, W2, b1r, b2r)
    xw = _tc_xw(xp, w_f)
    degp = _sc_degree(dst0, dst1).reshape(NC, NPAD, 16)
    h = _tc_scale(xw, degp)
    for hop in range(K_HOPS):
        pp = _sc_hop(h, src0, dst0, src1, dst1).reshape(NC, NPAD, DOUT)
        if hop < K_HOPS - 1:
            h = _tc_comb(pp, h, degp)
        else:
            h = _tc_final(pp, h, degp, bv)
    return h[:N]


# speedup vs baseline: 3.1477x; 1.2152x over previous
import functools

import jax
import jax.numpy as jnp
from jax import lax
from jax.experimental import pallas as pl
from jax.experimental.pallas import tpu as pltpu
from jax.experimental.pallas import tpu_sc as plsc

N = 10000
NPAD = 10240
E = 320000
DIN = 128
DOUT = 64
COLH = DOUT // 2
K_HOPS = 3

NC = 2
NS = 16
CH = 128
CPT = 160
EPAD = NS * CPT * CH
RPT = NPAD // NS
QR = RPT // 4

_mesh = plsc.VectorSubcoreMesh(core_axis_name="c", subcore_axis_name="s")
_sc_params = pltpu.CompilerParams(use_tc_tiling_on_sc=False,
                                  needs_layout_passes=False)


def _newton_rsqrt(x):
    i = plsc.bitcast(x, jnp.int32)
    i = jnp.int32(0x5F3759DF) - lax.shift_right_logical(i, 1)
    y = plsc.bitcast(i, jnp.float32)
    for _ in range(3):
        y = y * (1.5 - 0.5 * x * y * y)
    return y


@functools.partial(
    pl.kernel,
    out_type=(
        jax.ShapeDtypeStruct((NC * NPAD, COLH), jnp.float32),
        jax.ShapeDtypeStruct((NC * NPAD, COLH), jnp.float32),
        jax.ShapeDtypeStruct((NC * NPAD, COLH), jnp.float32),
    ),
    mesh=_mesh,
    scratch_types=[
        pltpu.VMEM((CPT, CH), jnp.int32),
        pltpu.VMEM((CPT, CH), jnp.int32),
        pltpu.VMEM((4, CH, COLH), jnp.float32),
        pltpu.VMEM((CH, 16), jnp.float32),
        pltpu.VMEM((RPT, 16), jnp.float32),
        pltpu.VMEM((QR, COLH), jnp.float32),
        pltpu.VMEM((QR, COLH), jnp.float32),
        pltpu.VMEM((QR, COLH), jnp.float32),
        pltpu.VMEM((QR, 16), jnp.float32),
        pltpu.VMEM((8, DOUT), jnp.float32),
        pltpu.VMEM_SHARED((NPAD, COLH), jnp.float32),
        pltpu.VMEM_SHARED((NPAD, 16), jnp.float32),
        pltpu.SemaphoreType.DMA,
        pltpu.SemaphoreType.DMA,
        pltpu.SemaphoreType.DMA,
        pltpu.SemaphoreType.DMA,
    ],
    compiler_params=_sc_params,
)
def _sc_mega(xw_hbm, srcr_hbm, dstr_hbm, bv_hbm,
             out_hbm, ping_hbm, pong_hbm,
             src_v, dst_v, gbuf, obuf, dbuf, astage, hstage, zbuf, zbuf16,
             bvv, acc_sp, deg_sp, sem0, sem1, sem2, sem3):
    cid = lax.axis_index("c")
    sid = lax.axis_index("s")
    base = sid * RPT
    gofs = cid * NPAD

    pltpu.sync_copy(srcr_hbm.at[sid], src_v)
    pltpu.sync_copy(dstr_hbm.at[sid], dst_v)
    pltpu.sync_copy(bv_hbm, bvv)

    goff_v = jnp.full((16,), gofs, dtype=jnp.int32)

    def add_off(r, _):
        for j in range(CH // 16):
            sl = pl.ds(j * 16, 16)
            src_v[r, sl] = src_v[r, sl] + goff_v
        return 0

    lax.fori_loop(0, CPT, add_off, 0)

    ones16 = jnp.full((16,), 1.0, dtype=jnp.float32)
    zeros16 = jnp.zeros((16,), jnp.float32)

    def fill_obuf(r, _):
        obuf[r, :] = ones16
        return 0

    lax.fori_loop(0, CH, fill_obuf, 0)

    def fill_z(r, _):
        zbuf16[r, :] = zeros16
        for j in range(COLH // 16):
            zbuf[r, pl.ds(j * 16, 16)] = zeros16
        return 0

    lax.fori_loop(0, QR, fill_z, 0)
    for q in range(4):
        pltpu.sync_copy(zbuf, acc_sp.at[pl.ds(base + q * QR, QR)])
        pltpu.sync_copy(zbuf16, deg_sp.at[pl.ds(base + q * QR, QR)])
    plsc.subcore_barrier()

    def dchunk(ci, _):
        pltpu.sync_copy(obuf, deg_sp.at[dst_v.at[ci]], add=True)
        return 0

    lax.fori_loop(0, CPT, dchunk, 0)
    plsc.subcore_barrier()
    pltpu.sync_copy(deg_sp.at[pl.ds(base, RPT)], dbuf)

    for q in range(4):
        sl = pl.ds(gofs + base + q * QR, QR)
        pltpu.sync_copy(xw_hbm.at[sl], hstage)

        def scale_row(r, _):
            deg = dbuf[q * QR + r, :] + 1.0
            s = _newton_rsqrt(deg)
            for j in range(COLH // 16):
                csl = pl.ds(j * 16, 16)
                hstage[r, csl] = hstage[r, csl] * s
            return 0

        lax.fori_loop(0, QR, scale_row, 0)
        pltpu.sync_copy(hstage, ping_hbm.at[sl])
    plsc.subcore_barrier()

    sems = (sem0, sem1, sem2, sem3)
    for hop in range(K_HOPS):
        cur = ping_hbm if hop % 2 == 0 else pong_hbm
        nxt = pong_hbm if hop % 2 == 0 else ping_hbm
        last = hop == K_HOPS - 1

        for b in range(4):
            pltpu.async_copy(cur.at[src_v.at[b]], gbuf.at[b], sems[b])

        def outer(j, _):
            for b in range(4):
                ci = j * 4 + b
                pltpu.make_async_copy(cur.at[src_v.at[ci]], gbuf.at[b],
                                      sems[b]).wait()
                pltpu.sync_copy(gbuf.at[b], acc_sp.at[dst_v.at[ci]],
                                add=True)

                @pl.when(ci + 4 < CPT)
                def _():
                    pltpu.async_copy(cur.at[src_v.at[ci + 4]], gbuf.at[b],
                                     sems[b])
            return 0

        lax.fori_loop(0, CPT // 4, outer, 0)
        plsc.subcore_barrier()

        for q in range(4):
            asl = pl.ds(base + q * QR, QR)
            gsl = pl.ds(gofs + base + q * QR, QR)
            pltpu.sync_copy(acc_sp.at[asl], astage)
            pltpu.sync_copy(cur.at[gsl], hstage)

            def comb_row(r, _):
                deg = dbuf[q * QR + r, :] + 1.0
                if last:
                    sc = _newton_rsqrt(deg)
                else:
                    sc = 1.0 / deg
                for j in range(COLH // 16):
                    csl = pl.ds(j * 16, 16)
                    v = (astage[r, csl] + hstage[r, csl]) * sc
                    if last:
                        v = v + bvv[0, pl.ds(cid * COLH + j * 16, 16)]
                    astage[r, csl] = v
                return 0

            lax.fori_loop(0, QR, comb_row, 0)
            dst_ref = out_hbm if last else nxt
            pltpu.sync_copy(astage, dst_ref.at[gsl])
            if not last:
                pltpu.sync_copy(zbuf, acc_sp.at[asl])
        if not last:
            plsc.subcore_barrier()


def _tc_w_body(w1_ref, w2_ref, b1_ref, b2_ref, w_ref, bv_ref):
    w_ref[...] = jnp.dot(w1_ref[...], w2_ref[...],
                         preferred_element_type=jnp.float32)
    bv_ref[...] = jnp.dot(b1_ref[...], w2_ref[...],
                          preferred_element_type=jnp.float32) + b2_ref[...]


_tc_w = pl.pallas_call(
    _tc_w_body,
    out_shape=(
        jax.ShapeDtypeStruct((DIN, DOUT), jnp.float32),
        jax.ShapeDtypeStruct((8, DOUT), jnp.float32),
    ),
)


def _tc_xw2_body(x_ref, w_ref, o_ref):
    o_ref[...] = jnp.dot(x_ref[...], w_ref[0],
                         preferred_element_type=jnp.float32)


_tc_xw2 = pl.pallas_call(
    _tc_xw2_body,
    grid=(NC, NPAD // 128),
    in_specs=[
        pl.BlockSpec((128, DIN), lambda c, i: (i, 0)),
        pl.BlockSpec((1, DIN, COLH), lambda c, i: (c, 0, 0)),
    ],
    out_specs=pl.BlockSpec((128, COLH), lambda c, i: (c * (NPAD // 128) + i, 0)),
    out_shape=jax.ShapeDtypeStruct((NC * NPAD, COLH), jnp.float32),
)


@jax.jit
def kernel(x, edge_index, W1, b1, W2, b2):
    src = edge_index[0].astype(jnp.int32)
    dst = edge_index[1].astype(jnp.int32)
    ar = jnp.arange(EPAD - E, dtype=jnp.int32)
    pad_src = ar % NPAD
    pad_dst = N + (ar % (NPAD - N))
    srcr = jnp.concatenate([src, pad_src]).reshape(NS, CPT, CH)
    dstr = jnp.concatenate([dst, pad_dst]).reshape(NS, CPT, CH)
    xp = jnp.pad(x, ((0, NPAD - N), (0, 0)))
    b1r = jnp.broadcast_to(b1[None, :], (8, DIN))
    b2r = jnp.broadcast_to(b2[None, :], (8, DOUT))

    w_f, bv = _tc_w(W1, W2, b1r, b2r)
    w_s = jnp.stack([w_f[:, :COLH], w_f[:, COLH:]])
    xw2 = _tc_xw2(xp, w_s)
    outf, _, _ = _sc_mega(xw2, srcr, dstr, bv)
    out = jnp.concatenate(
        [outf[:NPAD], outf[NPAD:]], axis=1)
    return out[:N]


# speedup vs baseline: 3.2616x; 1.0362x over previous
import functools

import jax
import jax.numpy as jnp
from jax import lax
from jax.experimental import pallas as pl
from jax.experimental.pallas import tpu as pltpu
from jax.experimental.pallas import tpu_sc as plsc

N = 10000
NPAD = 10240
E = 320000
DIN = 128
DOUT = 64
COLH = DOUT // 2
K_HOPS = 3

NC = 2
NS = 16
CH = 128
CPT = 160
EPAD = NS * CPT * CH
RPT = NPAD // NS
QR = RPT // 4

_mesh = plsc.VectorSubcoreMesh(core_axis_name="c", subcore_axis_name="s")
_sc_params = pltpu.CompilerParams(use_tc_tiling_on_sc=False,
                                  needs_layout_passes=False)


def _newton_rsqrt(x):
    i = plsc.bitcast(x, jnp.int32)
    i = jnp.int32(0x5F3759DF) - lax.shift_right_logical(i, 1)
    y = plsc.bitcast(i, jnp.float32)
    for _ in range(3):
        y = y * (1.5 - 0.5 * x * y * y)
    return y


@functools.partial(
    pl.kernel,
    out_type=(
        jax.ShapeDtypeStruct((NC * NPAD, COLH), jnp.float32),
        jax.ShapeDtypeStruct((NC * NPAD, COLH), jnp.float32),
        jax.ShapeDtypeStruct((NC * NPAD, COLH), jnp.float32),
    ),
    mesh=_mesh,
    scratch_types=[
        pltpu.VMEM((CPT, CH), jnp.int32),
        pltpu.VMEM((CPT, CH), jnp.int32),
        pltpu.VMEM((5, CH, COLH), jnp.float32),
        pltpu.VMEM((CH, 16), jnp.float32),
        pltpu.VMEM((RPT, 16), jnp.float32),
        pltpu.VMEM((QR, COLH), jnp.float32),
        pltpu.VMEM((QR, COLH), jnp.float32),
        pltpu.VMEM((QR, COLH), jnp.float32),
        pltpu.VMEM((QR, 16), jnp.float32),
        pltpu.VMEM((8, DOUT), jnp.float32),
        pltpu.VMEM_SHARED((NPAD, COLH), jnp.float32),
        pltpu.VMEM_SHARED((NPAD, 16), jnp.float32),
        pltpu.SemaphoreType.DMA,
        pltpu.SemaphoreType.DMA,
        pltpu.SemaphoreType.DMA,
        pltpu.SemaphoreType.DMA,
        pltpu.SemaphoreType.DMA,
    ],
    compiler_params=_sc_params,
)
def _sc_mega(xw_hbm, srcr_hbm, dstr_hbm, bv_hbm,
             out_hbm, ping_hbm, pong_hbm,
             src_v, dst_v, gbuf, obuf, dbuf, astage, hstage, zbuf, zbuf16,
             bvv, acc_sp, deg_sp,
             sem0, sem1, sem2, sem3, sem4):
    cid = lax.axis_index("c")
    sid = lax.axis_index("s")
    base = sid * RPT
    gofs = cid * NPAD

    pltpu.sync_copy(srcr_hbm.at[sid], src_v)
    pltpu.sync_copy(dstr_hbm.at[sid], dst_v)
    pltpu.sync_copy(bv_hbm, bvv)

    goff_v = jnp.full((16,), gofs, dtype=jnp.int32)

    def add_off(r, _):
        for j in range(CH // 16):
            sl = pl.ds(j * 16, 16)
            src_v[r, sl] = src_v[r, sl] + goff_v
        return 0

    lax.fori_loop(0, CPT, add_off, 0)

    ones16 = jnp.full((16,), 1.0, dtype=jnp.float32)
    zeros16 = jnp.zeros((16,), jnp.float32)

    def fill_obuf(r, _):
        obuf[r, :] = ones16
        return 0

    lax.fori_loop(0, CH, fill_obuf, 0)

    def fill_z(r, _):
        zbuf16[r, :] = zeros16
        for j in range(COLH // 16):
            zbuf[r, pl.ds(j * 16, 16)] = zeros16
        return 0

    lax.fori_loop(0, QR, fill_z, 0)
    for q in range(4):
        pltpu.sync_copy(zbuf, acc_sp.at[pl.ds(base + q * QR, QR)])
        pltpu.sync_copy(zbuf16, deg_sp.at[pl.ds(base + q * QR, QR)])
    plsc.subcore_barrier()

    def dchunk(ci, _):
        pltpu.sync_copy(obuf, deg_sp.at[dst_v.at[ci]], add=True)
        return 0

    lax.fori_loop(0, CPT, dchunk, 0)
    plsc.subcore_barrier()
    pltpu.sync_copy(deg_sp.at[pl.ds(base, RPT)], dbuf)

    for q in range(4):
        sl = pl.ds(gofs + base + q * QR, QR)
        pltpu.sync_copy(xw_hbm.at[sl], hstage)

        def scale_row(r, _):
            deg = dbuf[q * QR + r, :] + 1.0
            s = _newton_rsqrt(deg)
            for j in range(COLH // 16):
                csl = pl.ds(j * 16, 16)
                hstage[r, csl] = hstage[r, csl] * s
            return 0

        lax.fori_loop(0, QR, scale_row, 0)
        pltpu.sync_copy(hstage, ping_hbm.at[sl])
    plsc.subcore_barrier()

    sems = (sem0, sem1, sem2, sem3, sem4)
    NB = 5
    for hop in range(K_HOPS):
        cur = ping_hbm if hop % 2 == 0 else pong_hbm
        nxt = pong_hbm if hop % 2 == 0 else ping_hbm
        last = hop == K_HOPS - 1

        for b in range(NB):
            pltpu.async_copy(cur.at[src_v.at[b]], gbuf.at[b], sems[b])

        def outer(j, _):
            for b in range(NB):
                ci = j * NB + b
                pltpu.make_async_copy(cur.at[src_v.at[ci]], gbuf.at[b],
                                      sems[b]).wait()
                pltpu.sync_copy(gbuf.at[b], acc_sp.at[dst_v.at[ci]],
                                add=True)

                @pl.when(ci + NB < CPT)
                def _():
                    pltpu.async_copy(cur.at[src_v.at[ci + NB]], gbuf.at[b],
                                     sems[b])
            return 0

        lax.fori_loop(0, CPT // NB, outer, 0)
        plsc.subcore_barrier()

        for q in range(4):
            asl = pl.ds(base + q * QR, QR)
            gsl = pl.ds(gofs + base + q * QR, QR)
            pltpu.sync_copy(acc_sp.at[asl], astage)
            pltpu.sync_copy(cur.at[gsl], hstage)

            def comb_row(r, _):
                deg = dbuf[q * QR + r, :] + 1.0
                if last:
                    sc = _newton_rsqrt(deg)
                else:
                    sc = 1.0 / deg
                for j in range(COLH // 16):
                    csl = pl.ds(j * 16, 16)
                    v = (astage[r, csl] + hstage[r, csl]) * sc
                    if last:
                        v = v + bvv[0, pl.ds(cid * COLH + j * 16, 16)]
                    astage[r, csl] = v
                return 0

            lax.fori_loop(0, QR, comb_row, 0)
            dst_ref = out_hbm if last else nxt
            pltpu.sync_copy(astage, dst_ref.at[gsl])
            if not last:
                pltpu.sync_copy(zbuf, acc_sp.at[asl])
        if not last:
            plsc.subcore_barrier()


def _tc_w_body(w1_ref, w2_ref, b1_ref, b2_ref, w_ref, bv_ref):
    w_ref[...] = jnp.dot(w1_ref[...], w2_ref[...],
                         preferred_element_type=jnp.float32)
    bv_ref[...] = jnp.dot(b1_ref[...], w2_ref[...],
                          preferred_element_type=jnp.float32) + b2_ref[...]


_tc_w = pl.pallas_call(
    _tc_w_body,
    out_shape=(
        jax.ShapeDtypeStruct((DIN, DOUT), jnp.float32),
        jax.ShapeDtypeStruct((8, DOUT), jnp.float32),
    ),
)


def _tc_xw2_body(x_ref, w_ref, o_ref):
    o_ref[...] = jnp.dot(x_ref[...], w_ref[0],
                         preferred_element_type=jnp.float32)


_tc_xw2 = pl.pallas_call(
    _tc_xw2_body,
    grid=(NC, NPAD // 128),
    in_specs=[
        pl.BlockSpec((128, DIN), lambda c, i: (i, 0)),
        pl.BlockSpec((1, DIN, COLH), lambda c, i: (c, 0, 0)),
    ],
    out_specs=pl.BlockSpec((128, COLH), lambda c, i: (c * (NPAD // 128) + i, 0)),
    out_shape=jax.ShapeDtypeStruct((NC * NPAD, COLH), jnp.float32),
)


@jax.jit
def kernel(x, edge_index, W1, b1, W2, b2):
    src = edge_index[0].astype(jnp.int32)
    dst = edge_index[1].astype(jnp.int32)
    ar = jnp.arange(EPAD - E, dtype=jnp.int32)
    pad_src = ar % NPAD
    pad_dst = N + (ar % (NPAD - N))
    srcr = jnp.concatenate([src, pad_src]).reshape(NS, CPT, CH)
    dstr = jnp.concatenate([dst, pad_dst]).reshape(NS, CPT, CH)
    xp = jnp.pad(x, ((0, NPAD - N), (0, 0)))
    b1r = jnp.broadcast_to(b1[None, :], (8, DIN))
    b2r = jnp.broadcast_to(b2[None, :], (8, DOUT))

    w_f, bv = _tc_w(W1, W2, b1r, b2r)
    w_s = jnp.stack([w_f[:, :COLH], w_f[:, COLH:]])
    xw2 = _tc_xw2(xp, w_s)
    outf, _, _ = _sc_mega(xw2, srcr, dstr, bv)
    out = jnp.concatenate(
        [outf[:NPAD], outf[NPAD:]], axis=1)
    return out[:N]
